# baseline (speedup 1.0000x reference)
import jax
import jax.numpy as jnp
from jax import lax
from jax.experimental import pallas as pl
from jax.experimental.pallas import tpu as pltpu
from jax.experimental.pallas import tpu_sc as plsc

N_NODES = 10000
N_PAD = 10240
D = 128
N_EDGES = 320000
NC, NS, L = 2, 16, 16
NW = NC * NS
W = 128
WPW = 80
EPW = W * WPW
E_PAD = EPW * NW
ROWS_PER_SUB = N_PAD // NS

_MESH = plsc.VectorSubcoreMesh(
    core_axis_name="c", subcore_axis_name="s", num_cores=NC, num_subcores=NS
)


def _deg_body(dst_hbm, out_hbm, dstv, hist):
    wid = lax.axis_index("s") * NC + lax.axis_index("c")

    @pl.loop(0, N_PAD // L)
    def _(i):
        hist[pl.ds(i * L, L)] = jnp.zeros((L,), jnp.float32)

    pltpu.sync_copy(dst_hbm.at[pl.ds(wid * EPW, EPW)], dstv)
    ones = jnp.ones((L,), jnp.float32)

    @pl.loop(0, EPW // L)
    def _(i):
        idx = dstv[pl.ds(i * L, L)]
        plsc.addupdate_scatter(hist, [idx], ones)

    pltpu.sync_copy(hist, out_hbm.at[wid])


_deg_kernel = pl.kernel(
    _deg_body,
    out_type=jax.ShapeDtypeStruct((NW, N_PAD), jnp.float32),
    mesh=_MESH,
    scratch_types=[
        pltpu.VMEM((EPW,), jnp.int32),
        pltpu.VMEM((N_PAD,), jnp.float32),
    ],
    compiler_params=pltpu.CompilerParams(needs_layout_passes=False),
)


HALF = WPW // 2


def _scat_body(g_hbm, src_hbm, dst_hbm, zeros_hbm, out_hbm, acc, srcv, dstv,
               rA, rB, sA, sB, wA, wB):
    c = lax.axis_index("c")
    s = lax.axis_index("s")
    wid = s * NC + c

    @pl.loop(0, ROWS_PER_SUB // W)
    def _(k):
        pltpu.sync_copy(zeros_hbm, acc.at[pl.ds(s * ROWS_PER_SUB + k * W, W)])

    plsc.subcore_barrier()

    for p in range(2):
        pltpu.sync_copy(src_hbm.at[wid].at[pl.ds(p * HALF, HALF)], srcv)
        pltpu.sync_copy(dst_hbm.at[wid].at[pl.ds(p * HALF, HALF)], dstv)
        pltpu.async_copy(g_hbm.at[srcv.at[0]], rA, sA)
        pltpu.async_copy(g_hbm.at[srcv.at[1]], rB, sB)

        @pl.loop(0, HALF, step=2)
        def _(j):
            pltpu.make_async_copy(g_hbm.at[srcv.at[j]], rA, sA).wait()
            pltpu.async_copy(rA, acc.at[dstv.at[j]], wA, add=True)
            pltpu.make_async_copy(g_hbm.at[srcv.at[j + 1]], rB, sB).wait()
            pltpu.async_copy(rB, acc.at[dstv.at[j + 1]], wB, add=True)

            pltpu.make_async_copy(rA, acc.at[dstv.at[j]], wA).wait()

            @pl.when(j + 2 < HALF)
            def _():
                pltpu.async_copy(g_hbm.at[srcv.at[j + 2]], rA, sA)

            pltpu.make_async_copy(rB, acc.at[dstv.at[j + 1]], wB).wait()

            @pl.when(j + 3 < HALF)
            def _():
                pltpu.async_copy(g_hbm.at[srcv.at[j + 3]], rB, sB)

    plsc.subcore_barrier()

    @pl.loop(0, ROWS_PER_SUB // W)
    def _(k):
        r0 = s * ROWS_PER_SUB + k * W
        pltpu.sync_copy(acc.at[pl.ds(r0, W)], out_hbm.at[c].at[pl.ds(r0, W)])


_scat_kernel = pl.kernel(
    _scat_body,
    out_type=jax.ShapeDtypeStruct((NC, N_PAD, D), jnp.float32),
    mesh=_MESH,
    scratch_types=[
        pltpu.VMEM_SHARED((N_PAD, D), jnp.float32),
        pltpu.VMEM((HALF, W), jnp.int32),
        pltpu.VMEM((HALF, W), jnp.int32),
        pltpu.VMEM((W, D), jnp.float32),
        pltpu.VMEM((W, D), jnp.float32),
        pltpu.SemaphoreType.DMA,
        pltpu.SemaphoreType.DMA,
        pltpu.SemaphoreType.DMA,
        pltpu.SemaphoreType.DMA,
    ],
)


def _tc0_body(x_ref, w1_ref, h_ref):
    h_ref[...] = jnp.dot(
        x_ref[...], w1_ref[...],
        preferred_element_type=jnp.float32, precision=lax.Precision.HIGHEST,
    )


_tc0 = pl.pallas_call(
    _tc0_body,
    out_shape=jax.ShapeDtypeStruct((N_NODES, D), jnp.float32),
)


def _tc1_body(degp_ref, h_ref, g1_ref, d_ref):
    deg = jnp.sum(degp_ref[:, :N_NODES], axis=0) + 1.0
    dis = lax.rsqrt(deg)[:, None]
    g1_ref[...] = h_ref[...] * dis
    d_ref[...] = dis


_tc1 = pl.pallas_call(
    _tc1_body,
    out_shape=(
        jax.ShapeDtypeStruct((N_NODES, D), jnp.float32),
        jax.ShapeDtypeStruct((N_NODES, 1), jnp.float32),
    ),
)


def _tc2_body(sp_ref, g1_ref, d_ref, b1_ref, w2_ref, g2_ref):
    ssum = sp_ref[0, :N_NODES, :] + sp_ref[1, :N_NODES, :] + g1_ref[...]
    out1 = jnp.maximum(d_ref[...] * ssum + b1_ref[...], 0.0)
    h2 = jnp.dot(
        out1, w2_ref[...],
        preferred_element_type=jnp.float32, precision=lax.Precision.HIGHEST,
    )
    g2_ref[...] = h2 * d_ref[...]


_tc2 = pl.pallas_call(
    _tc2_body,
    out_shape=jax.ShapeDtypeStruct((N_NODES, D), jnp.float32),
)


def _tc3_body(sp_ref, g2_ref, d_ref, b2_ref, o_ref):
    ssum = sp_ref[0, :N_NODES, :] + sp_ref[1, :N_NODES, :] + g2_ref[...]
    o_ref[...] = d_ref[...] * ssum + b2_ref[...]


_tc3 = pl.pallas_call(
    _tc3_body,
    out_shape=jax.ShapeDtypeStruct((N_NODES, D), jnp.float32),
)


def kernel(x, edge_index, W1, b1, W2, b2):
    ei = edge_index.astype(jnp.int32)
    src, dst = ei[0], ei[1]
    pad_n = E_PAD - N_EDGES
    pad_iota = jnp.arange(pad_n, dtype=jnp.int32)
    src_p = jnp.concatenate([src, pad_iota % N_NODES])
    dst_p = jnp.concatenate([dst, N_NODES + pad_iota % (N_PAD - N_NODES)])
    src3 = src_p.reshape(NW, WPW, W)
    dst3 = dst_p.reshape(NW, WPW, W)
    zeros = jnp.zeros((W, D), jnp.float32)

    degp = _deg_kernel(dst_p)
    h1 = _tc0(x, W1)
    g1, d = _tc1(degp, h1)
    s1 = _scat_kernel(g1, src3, dst3, zeros)
    g2 = _tc2(s1, g1, d, b1.reshape(1, D), W2)
    s2 = _scat_kernel(g2, src3, dst3, zeros)
    return _tc3(s2, g2, d, b2.reshape(1, D))


# speedup vs baseline: 1.2278x; 1.2278x over previous
import jax
import jax.numpy as jnp
from jax import lax
from jax.experimental import pallas as pl
from jax.experimental.pallas import tpu as pltpu
from jax.experimental.pallas import tpu_sc as plsc

N_NODES = 10000
N_PAD = 10240
D = 128
N_EDGES = 320000
NC, NS, L = 2, 16, 16
NW = NC * NS
W = 128
WPW = 80
EPW = W * WPW
E_PAD = EPW * NW
ROWS_PER_SUB = N_PAD // NS

_MESH = plsc.VectorSubcoreMesh(
    core_axis_name="c", subcore_axis_name="s", num_cores=NC, num_subcores=NS
)


def _deg_body(dst_hbm, out_hbm, dstv, hist):
    wid = lax.axis_index("s") * NC + lax.axis_index("c")

    @pl.loop(0, N_PAD // L)
    def _(i):
        hist[pl.ds(i * L, L)] = jnp.zeros((L,), jnp.float32)

    pltpu.sync_copy(dst_hbm.at[pl.ds(wid * EPW, EPW)], dstv)
    ones = jnp.ones((L,), jnp.float32)

    @pl.loop(0, EPW // L)
    def _(i):
        idx = dstv[pl.ds(i * L, L)]
        plsc.addupdate_scatter(hist, [idx], ones)

    pltpu.sync_copy(hist, out_hbm.at[wid])


_deg_kernel = pl.kernel(
    _deg_body,
    out_type=jax.ShapeDtypeStruct((NW, N_PAD), jnp.float32),
    mesh=_MESH,
    scratch_types=[
        pltpu.VMEM((EPW,), jnp.int32),
        pltpu.VMEM((N_PAD,), jnp.float32),
    ],
    compiler_params=pltpu.CompilerParams(needs_layout_passes=False),
)


HALF = WPW // 2


def _scat_body(g_hbm, src_hbm, dst_hbm, zeros_hbm, out_hbm, acc, srcv, dstv,
               rA, rB, sA, sB, wA, wB):
    c = lax.axis_index("c")
    s = lax.axis_index("s")
    wid = s * NC + c

    @pl.loop(0, ROWS_PER_SUB // W)
    def _(k):
        pltpu.sync_copy(zeros_hbm, acc.at[pl.ds(s * ROWS_PER_SUB + k * W, W)])

    plsc.subcore_barrier()

    for p in range(2):
        pltpu.sync_copy(src_hbm.at[wid].at[pl.ds(p * HALF, HALF)], srcv)
        pltpu.sync_copy(dst_hbm.at[wid].at[pl.ds(p * HALF, HALF)], dstv)
        pltpu.async_copy(g_hbm.at[srcv.at[0]], rA, sA)
        pltpu.async_copy(g_hbm.at[srcv.at[1]], rB, sB)

        @pl.loop(0, HALF, step=2)
        def _(j):
            pltpu.make_async_copy(g_hbm.at[srcv.at[j]], rA, sA).wait()
            pltpu.sync_copy(rA, acc.at[dstv.at[j]], add=True)

            @pl.when(j + 2 < HALF)
            def _():
                pltpu.async_copy(g_hbm.at[srcv.at[j + 2]], rA, sA)

            pltpu.make_async_copy(g_hbm.at[srcv.at[j + 1]], rB, sB).wait()
            pltpu.sync_copy(rB, acc.at[dstv.at[j + 1]], add=True)

            @pl.when(j + 3 < HALF)
            def _():
                pltpu.async_copy(g_hbm.at[srcv.at[j + 3]], rB, sB)

    plsc.subcore_barrier()

    @pl.loop(0, ROWS_PER_SUB // W)
    def _(k):
        r0 = s * ROWS_PER_SUB + k * W
        pltpu.sync_copy(acc.at[pl.ds(r0, W)], out_hbm.at[c].at[pl.ds(r0, W)])


_scat_kernel = pl.kernel(
    _scat_body,
    out_type=jax.ShapeDtypeStruct((NC, N_PAD, D), jnp.float32),
    mesh=_MESH,
    scratch_types=[
        pltpu.VMEM_SHARED((N_PAD, D), jnp.float32),
        pltpu.VMEM((HALF, W), jnp.int32),
        pltpu.VMEM((HALF, W), jnp.int32),
        pltpu.VMEM((W, D), jnp.float32),
        pltpu.VMEM((W, D), jnp.float32),
        pltpu.SemaphoreType.DMA,
        pltpu.SemaphoreType.DMA,
        pltpu.SemaphoreType.DMA,
        pltpu.SemaphoreType.DMA,
    ],
)


def _tc0_body(x_ref, w1_ref, h_ref):
    h_ref[...] = jnp.dot(
        x_ref[...], w1_ref[...],
        preferred_element_type=jnp.float32, precision=lax.Precision.HIGHEST,
    )


_tc0 = pl.pallas_call(
    _tc0_body,
    out_shape=jax.ShapeDtypeStruct((N_NODES, D), jnp.float32),
)


def _tc1_body(degp_ref, h_ref, g1_ref, d_ref):
    deg = jnp.sum(degp_ref[:, :N_NODES], axis=0) + 1.0
    dis = lax.rsqrt(deg)[:, None]
    g1_ref[...] = h_ref[...] * dis
    d_ref[...] = dis


_tc1 = pl.pallas_call(
    _tc1_body,
    out_shape=(
        jax.ShapeDtypeStruct((N_NODES, D), jnp.float32),
        jax.ShapeDtypeStruct((N_NODES, 1), jnp.float32),
    ),
)


def _tc2_body(sp_ref, g1_ref, d_ref, b1_ref, w2_ref, g2_ref):
    ssum = sp_ref[0, :N_NODES, :] + sp_ref[1, :N_NODES, :] + g1_ref[...]
    out1 = jnp.maximum(d_ref[...] * ssum + b1_ref[...], 0.0)
    h2 = jnp.dot(
        out1, w2_ref[...],
        preferred_element_type=jnp.float32, precision=lax.Precision.HIGHEST,
    )
    g2_ref[...] = h2 * d_ref[...]


_tc2 = pl.pallas_call(
    _tc2_body,
    out_shape=jax.ShapeDtypeStruct((N_NODES, D), jnp.float32),
)


def _tc3_body(sp_ref, g2_ref, d_ref, b2_ref, o_ref):
    ssum = sp_ref[0, :N_NODES, :] + sp_ref[1, :N_NODES, :] + g2_ref[...]
    o_ref[...] = d_ref[...] * ssum + b2_ref[...]


_tc3 = pl.pallas_call(
    _tc3_body,
    out_shape=jax.ShapeDtypeStruct((N_NODES, D), jnp.float32),
)


def kernel(x, edge_index, W1, b1, W2, b2):
    ei = edge_index.astype(jnp.int32)
    src, dst = ei[0], ei[1]
    pad_n = E_PAD - N_EDGES
    pad_iota = jnp.arange(pad_n, dtype=jnp.int32)
    src_p = jnp.concatenate([src, pad_iota % N_NODES])
    dst_p = jnp.concatenate([dst, N_NODES + pad_iota % (N_PAD - N_NODES)])
    src3 = src_p.reshape(NW, WPW, W)
    dst3 = dst_p.reshape(NW, WPW, W)
    zeros = jnp.zeros((W, D), jnp.float32)

    degp = _deg_kernel(dst_p)
    h1 = _tc0(x, W1)
    g1, d = _tc1(degp, h1)
    s1 = _scat_kernel(g1, src3, dst3, zeros)
    g2 = _tc2(s1, g1, d, b1.reshape(1, D), W2)
    s2 = _scat_kernel(g2, src3, dst3, zeros)
    return _tc3(s2, g2, d, b2.reshape(1, D))


# speedup vs baseline: 1.2866x; 1.0479x over previous
import jax
import jax.numpy as jnp
from jax import lax
from jax.experimental import pallas as pl
from jax.experimental.pallas import tpu as pltpu
from jax.experimental.pallas import tpu_sc as plsc

N_NODES = 10000
N_PAD = 10240
D = 128
N_EDGES = 320000
NC, NS, L = 2, 16, 16
NW = NC * NS
W = 128
WPW = 80
EPW = W * WPW
E_PAD = EPW * NW
ROWS_PER_SUB = N_PAD // NS

_MESH = plsc.VectorSubcoreMesh(
    core_axis_name="c", subcore_axis_name="s", num_cores=NC, num_subcores=NS
)


def _deg_body(dst_hbm, out_hbm, dstv, hist):
    wid = lax.axis_index("s") * NC + lax.axis_index("c")

    @pl.loop(0, N_PAD // L)
    def _(i):
        hist[pl.ds(i * L, L)] = jnp.zeros((L,), jnp.float32)

    pltpu.sync_copy(dst_hbm.at[pl.ds(wid * EPW, EPW)], dstv)
    ones = jnp.ones((L,), jnp.float32)

    @pl.loop(0, EPW // L)
    def _(i):
        idx = dstv[pl.ds(i * L, L)]
        plsc.addupdate_scatter(hist, [idx], ones)

    pltpu.sync_copy(hist, out_hbm.at[wid])


_deg_kernel = pl.kernel(
    _deg_body,
    out_type=jax.ShapeDtypeStruct((NW, N_PAD), jnp.float32),
    mesh=_MESH,
    scratch_types=[
        pltpu.VMEM((EPW,), jnp.int32),
        pltpu.VMEM((N_PAD,), jnp.float32),
    ],
    compiler_params=pltpu.CompilerParams(needs_layout_passes=False),
)


HALF = WPW // 2


def _scat_body(g_hbm, src_hbm, dst_hbm, zeros_hbm, out_hbm, acc, srcv, dstv,
               rA, rB, sA, sB, wA, wB):
    c = lax.axis_index("c")
    s = lax.axis_index("s")
    wid = s * NC + c

    pltpu.async_copy(zeros_hbm, acc.at[pl.ds(s * ROWS_PER_SUB, ROWS_PER_SUB)], wA)
    pltpu.async_copy(src_hbm.at[wid].at[pl.ds(0, HALF)], srcv, wB)
    pltpu.async_copy(dst_hbm.at[wid].at[pl.ds(0, HALF)], dstv, wB)
    pltpu.make_async_copy(src_hbm.at[wid].at[pl.ds(0, HALF)], srcv, wB).wait()
    pltpu.make_async_copy(dst_hbm.at[wid].at[pl.ds(0, HALF)], dstv, wB).wait()
    pltpu.async_copy(g_hbm.at[srcv.at[0]], rA, sA)
    pltpu.async_copy(g_hbm.at[srcv.at[1]], rB, sB)
    pltpu.make_async_copy(zeros_hbm, acc.at[pl.ds(s * ROWS_PER_SUB, ROWS_PER_SUB)], wA).wait()
    plsc.subcore_barrier()

    for p in range(2):
        if p:
            pltpu.sync_copy(src_hbm.at[wid].at[pl.ds(p * HALF, HALF)], srcv)
            pltpu.sync_copy(dst_hbm.at[wid].at[pl.ds(p * HALF, HALF)], dstv)
            pltpu.async_copy(g_hbm.at[srcv.at[0]], rA, sA)
            pltpu.async_copy(g_hbm.at[srcv.at[1]], rB, sB)

        @pl.loop(0, HALF, step=2)
        def _(j):
            pltpu.make_async_copy(g_hbm.at[srcv.at[j]], rA, sA).wait()
            pltpu.sync_copy(rA, acc.at[dstv.at[j]], add=True)

            @pl.when(j + 2 < HALF)
            def _():
                pltpu.async_copy(g_hbm.at[srcv.at[j + 2]], rA, sA)

            pltpu.make_async_copy(g_hbm.at[srcv.at[j + 1]], rB, sB).wait()
            pltpu.sync_copy(rB, acc.at[dstv.at[j + 1]], add=True)

            @pl.when(j + 3 < HALF)
            def _():
                pltpu.async_copy(g_hbm.at[srcv.at[j + 3]], rB, sB)

    plsc.subcore_barrier()
    r0 = s * ROWS_PER_SUB
    pltpu.sync_copy(acc.at[pl.ds(r0, ROWS_PER_SUB)],
                    out_hbm.at[c].at[pl.ds(r0, ROWS_PER_SUB)])


_scat_kernel = pl.kernel(
    _scat_body,
    out_type=jax.ShapeDtypeStruct((NC, N_PAD, D), jnp.float32),
    mesh=_MESH,
    scratch_types=[
        pltpu.VMEM_SHARED((N_PAD, D), jnp.float32),
        pltpu.VMEM((HALF, W), jnp.int32),
        pltpu.VMEM((HALF, W), jnp.int32),
        pltpu.VMEM((W, D), jnp.float32),
        pltpu.VMEM((W, D), jnp.float32),
        pltpu.SemaphoreType.DMA,
        pltpu.SemaphoreType.DMA,
        pltpu.SemaphoreType.DMA,
        pltpu.SemaphoreType.DMA,
    ],
)


def _tc1_body(degp_ref, x_ref, w1_ref, g1_ref, d_ref):
    deg = jnp.sum(degp_ref[:, :N_NODES], axis=0) + 1.0
    dis = lax.rsqrt(deg)[:, None]
    h = jnp.dot(
        x_ref[...], w1_ref[...],
        preferred_element_type=jnp.float32, precision=lax.Precision.HIGHEST,
    )
    g1_ref[...] = h * dis
    d_ref[...] = dis


_tc1 = pl.pallas_call(
    _tc1_body,
    out_shape=(
        jax.ShapeDtypeStruct((N_NODES, D), jnp.float32),
        jax.ShapeDtypeStruct((N_NODES, 1), jnp.float32),
    ),
)


def _tc2_body(sp_ref, g1_ref, d_ref, b1_ref, w2_ref, g2_ref):
    ssum = sp_ref[0, :N_NODES, :] + sp_ref[1, :N_NODES, :] + g1_ref[...]
    out1 = jnp.maximum(d_ref[...] * ssum + b1_ref[...], 0.0)
    h2 = jnp.dot(
        out1, w2_ref[...],
        preferred_element_type=jnp.float32, precision=lax.Precision.HIGHEST,
    )
    g2_ref[...] = h2 * d_ref[...]


_tc2 = pl.pallas_call(
    _tc2_body,
    out_shape=jax.ShapeDtypeStruct((N_NODES, D), jnp.float32),
)


def _tc3_body(sp_ref, g2_ref, d_ref, b2_ref, o_ref):
    ssum = sp_ref[0, :N_NODES, :] + sp_ref[1, :N_NODES, :] + g2_ref[...]
    o_ref[...] = d_ref[...] * ssum + b2_ref[...]


_tc3 = pl.pallas_call(
    _tc3_body,
    out_shape=jax.ShapeDtypeStruct((N_NODES, D), jnp.float32),
)


def kernel(x, edge_index, W1, b1, W2, b2):
    ei = edge_index.astype(jnp.int32)
    src, dst = ei[0], ei[1]
    pad_n = E_PAD - N_EDGES
    pad_iota = jnp.arange(pad_n, dtype=jnp.int32)
    src_p = jnp.concatenate([src, pad_iota % N_NODES])
    dst_p = jnp.concatenate([dst, N_NODES + pad_iota % (N_PAD - N_NODES)])
    src3 = src_p.reshape(NW, WPW, W)
    dst3 = dst_p.reshape(NW, WPW, W)
    zeros = jnp.zeros((ROWS_PER_SUB, D), jnp.float32)

    degp = _deg_kernel(dst_p)
    g1, d = _tc1(degp, x, W1)
    s1 = _scat_kernel(g1, src3, dst3, zeros)
    g2 = _tc2(s1, g1, d, b1.reshape(1, D), W2)
    s2 = _scat_kernel(g2, src3, dst3, zeros)
    return _tc3(s2, g2, d, b2.reshape(1, D))


# speedup vs baseline: 1.3541x; 1.0524x over previous
import jax
import jax.numpy as jnp
from jax import lax
from jax.experimental import pallas as pl
from jax.experimental.pallas import tpu as pltpu
from jax.experimental.pallas import tpu_sc as plsc

N_NODES = 10000
N_PAD = 10240
D = 128
N_EDGES = 320000
NC, NS, L = 2, 16, 16
NW = NC * NS
W = 64
WPW = 160
EPW = W * WPW
E_PAD = EPW * NW
ROWS_PER_SUB = N_PAD // NS

_MESH = plsc.VectorSubcoreMesh(
    core_axis_name="c", subcore_axis_name="s", num_cores=NC, num_subcores=NS
)


def _deg_body(dst_hbm, out_hbm, dstv, hist):
    wid = lax.axis_index("s") * NC + lax.axis_index("c")

    @pl.loop(0, N_PAD // L)
    def _(i):
        hist[pl.ds(i * L, L)] = jnp.zeros((L,), jnp.float32)

    pltpu.sync_copy(dst_hbm.at[pl.ds(wid * EPW, EPW)], dstv)
    ones = jnp.ones((L,), jnp.float32)

    @pl.loop(0, EPW // L)
    def _(i):
        idx = dstv[pl.ds(i * L, L)]
        plsc.addupdate_scatter(hist, [idx], ones)

    pltpu.sync_copy(hist, out_hbm.at[wid])


_deg_kernel = pl.kernel(
    _deg_body,
    out_type=jax.ShapeDtypeStruct((NW, N_PAD), jnp.float32),
    mesh=_MESH,
    scratch_types=[
        pltpu.VMEM((EPW,), jnp.int32),
        pltpu.VMEM((N_PAD,), jnp.float32),
    ],
    compiler_params=pltpu.CompilerParams(needs_layout_passes=False),
)


HALF = WPW // 4


NBUF = 4


def _scat_body(g_hbm, src_hbm, dst_hbm, zeros_hbm, out_hbm, acc, srcv, dstv,
               r0b, r1b, r2b, r3b, s0, s1, s2, s3, wA, wB):
    c = lax.axis_index("c")
    s = lax.axis_index("s")
    wid = s * NC + c
    bufs = (r0b, r1b, r2b, r3b)
    sems = (s0, s1, s2, s3)

    pltpu.async_copy(zeros_hbm, acc.at[pl.ds(s * ROWS_PER_SUB, ROWS_PER_SUB)], wA)
    pltpu.async_copy(src_hbm.at[wid].at[pl.ds(0, HALF)], srcv, wB)
    pltpu.async_copy(dst_hbm.at[wid].at[pl.ds(0, HALF)], dstv, wB)
    pltpu.make_async_copy(src_hbm.at[wid].at[pl.ds(0, HALF)], srcv, wB).wait()
    pltpu.make_async_copy(dst_hbm.at[wid].at[pl.ds(0, HALF)], dstv, wB).wait()
    for b in range(NBUF):
        pltpu.async_copy(g_hbm.at[srcv.at[b]], bufs[b], sems[b])
    pltpu.make_async_copy(zeros_hbm, acc.at[pl.ds(s * ROWS_PER_SUB, ROWS_PER_SUB)], wA).wait()
    plsc.subcore_barrier()

    for p in range(4):
        if p:
            pltpu.sync_copy(src_hbm.at[wid].at[pl.ds(p * HALF, HALF)], srcv)
            pltpu.sync_copy(dst_hbm.at[wid].at[pl.ds(p * HALF, HALF)], dstv)
            for b in range(NBUF):
                pltpu.async_copy(g_hbm.at[srcv.at[b]], bufs[b], sems[b])

        @pl.loop(0, HALF, step=NBUF)
        def _(j):
            for b in range(NBUF):
                jj = j + b
                pltpu.make_async_copy(g_hbm.at[srcv.at[jj]], bufs[b], sems[b]).wait()
                pltpu.sync_copy(bufs[b], acc.at[dstv.at[jj]], add=True)

                @pl.when(jj + NBUF < HALF)
                def _():
                    pltpu.async_copy(g_hbm.at[srcv.at[jj + NBUF]], bufs[b], sems[b])

    plsc.subcore_barrier()
    r0 = s * ROWS_PER_SUB
    pltpu.sync_copy(acc.at[pl.ds(r0, ROWS_PER_SUB)],
                    out_hbm.at[c].at[pl.ds(r0, ROWS_PER_SUB)])


_scat_kernel = pl.kernel(
    _scat_body,
    out_type=jax.ShapeDtypeStruct((NC, N_PAD, D), jnp.float32),
    mesh=_MESH,
    scratch_types=[
        pltpu.VMEM_SHARED((N_PAD, D), jnp.float32),
        pltpu.VMEM((HALF, W), jnp.int32),
        pltpu.VMEM((HALF, W), jnp.int32),
        pltpu.VMEM((W, D), jnp.float32),
        pltpu.VMEM((W, D), jnp.float32),
        pltpu.VMEM((W, D), jnp.float32),
        pltpu.VMEM((W, D), jnp.float32),
        pltpu.SemaphoreType.DMA,
        pltpu.SemaphoreType.DMA,
        pltpu.SemaphoreType.DMA,
        pltpu.SemaphoreType.DMA,
        pltpu.SemaphoreType.DMA,
        pltpu.SemaphoreType.DMA,
    ],
)


def _tc1_body(degp_ref, x_ref, w1_ref, g1_ref, d_ref):
    deg = jnp.sum(degp_ref[:, :N_NODES], axis=0) + 1.0
    dis = lax.rsqrt(deg)[:, None]
    h = jnp.dot(
        x_ref[...], w1_ref[...],
        preferred_element_type=jnp.float32, precision=lax.Precision.HIGHEST,
    )
    g1_ref[...] = h * dis
    d_ref[...] = dis


_tc1 = pl.pallas_call(
    _tc1_body,
    out_shape=(
        jax.ShapeDtypeStruct((N_NODES, D), jnp.float32),
        jax.ShapeDtypeStruct((N_NODES, 1), jnp.float32),
    ),
)


def _tc2_body(sp_ref, g1_ref, d_ref, b1_ref, w2_ref, g2_ref):
    ssum = sp_ref[0, :N_NODES, :] + sp_ref[1, :N_NODES, :] + g1_ref[...]
    out1 = jnp.maximum(d_ref[...] * ssum + b1_ref[...], 0.0)
    h2 = jnp.dot(
        out1, w2_ref[...],
        preferred_element_type=jnp.float32, precision=lax.Precision.HIGHEST,
    )
    g2_ref[...] = h2 * d_ref[...]


_tc2 = pl.pallas_call(
    _tc2_body,
    out_shape=jax.ShapeDtypeStruct((N_NODES, D), jnp.float32),
)


def _tc3_body(sp_ref, g2_ref, d_ref, b2_ref, o_ref):
    ssum = sp_ref[0, :N_NODES, :] + sp_ref[1, :N_NODES, :] + g2_ref[...]
    o_ref[...] = d_ref[...] * ssum + b2_ref[...]


_tc3 = pl.pallas_call(
    _tc3_body,
    out_shape=jax.ShapeDtypeStruct((N_NODES, D), jnp.float32),
)


def kernel(x, edge_index, W1, b1, W2, b2):
    ei = edge_index.astype(jnp.int32)
    src, dst = ei[0], ei[1]
    pad_n = E_PAD - N_EDGES
    pad_iota = jnp.arange(pad_n, dtype=jnp.int32)
    src_p = jnp.concatenate([src, pad_iota % N_NODES])
    dst_p = jnp.concatenate([dst, N_NODES + pad_iota % (N_PAD - N_NODES)])
    src3 = src_p.reshape(NW, WPW, W)
    dst3 = dst_p.reshape(NW, WPW, W)
    zeros = jnp.zeros((ROWS_PER_SUB, D), jnp.float32)

    degp = _deg_kernel(dst_p)
    g1, d = _tc1(degp, x, W1)
    s1 = _scat_kernel(g1, src3, dst3, zeros)
    g2 = _tc2(s1, g1, d, b1.reshape(1, D), W2)
    s2 = _scat_kernel(g2, src3, dst3, zeros)
    return _tc3(s2, g2, d, b2.reshape(1, D))
